# Initial kernel scaffold; baseline (speedup 1.0000x reference)
import jax
import jax.numpy as jnp
from jax.experimental import pallas as pl


def kernel(x_enc, x_mark_enc, W_patch, W_seq, pos_emb, ln1_s, ln1_b, Wqkv, Wo, ln2_s, ln2_b, Wr, We1, We2, Wcls, bcls, cat_tok):
    raise NotImplementedError("write your pallas kernel here")


# speedup vs baseline: 1.0133x; 1.0133x over previous
import functools

import numpy as np
import jax
import jax.numpy as jnp
from jax import lax
from jax.experimental import pallas as pl

B = 2; T = 2048; V = 8; P = 64; NPATCH = T // P; C = 768; H = 12; DH = C // H
LAYERS = 2; NEXP = 8; TOPK = 2; DFF = 3072; K = 10; L = NPATCH + 1
FREQ_P = P // 2 + 1; FREQ_S = T // 2 + 1
NSEQ = B * V
LP = 40
NTOK = NSEQ * LP


def _dft_mats(n, nfreq):
    ns = np.arange(n)[:, None]
    ks = np.arange(nfreq)[None, :]
    ang = 2.0 * np.pi * ns * ks / n
    return np.cos(ang).astype(np.float32), np.sin(ang).astype(np.float32)

_DPC, _DPS = _dft_mats(P, FREQ_P)
_DSC, _DSS = _dft_mats(T, FREQ_S)


def _ln_in(x, s, b):
    m = jnp.mean(x, axis=-1, keepdims=True)
    v = jnp.mean((x - m) * (x - m), axis=-1, keepdims=True)
    return (x - m) * lax.rsqrt(v + 1e-6) * s + b


def _dot(a, b):
    return jnp.dot(a, b, preferred_element_type=jnp.float32)


def _dot_t(a, b):
    return lax.dot_general(a, b, (((1,), (1,)), ((), ())),
                           preferred_element_type=jnp.float32)


def _tokenize_kernel(patches_ref, x_ref, dpc_ref, dps_ref, wp_ref,
                     dsc_ref, dss_ref, ws_ref, pos_ref, h_ref):
    pr = patches_ref[...]
    re = _dot(pr, dpc_ref[...])
    im = _dot(pr, dps_ref[...])
    pf = jnp.sqrt(re * re + im * im)
    tok = _dot(pf, wp_ref[...])
    xr = x_ref[...]
    sre = _dot(xr, dsc_ref[...])
    sim_ = _dot(xr, dss_ref[...])
    sf = jnp.sqrt(sre * sre + sim_ * sim_)
    cls = _dot(sf, ws_ref[...])
    pos = pos_ref[...]
    zero = jnp.zeros((LP - L, C), jnp.float32)
    for s in range(NSEQ):
        h_ref[s * LP:s * LP + 1, :] = cls[s:s + 1, :] + pos[0:1, :]
        h_ref[s * LP + 1:s * LP + L, :] = tok[s * NPATCH:(s + 1) * NPATCH, :] + pos[1:L, :]
        h_ref[s * LP + L:(s + 1) * LP, :] = zero


def _attn_kernel(h_ref, s_ref, b_ref, wqkv_ref, wo_ref, out_ref):
    hs = h_ref[0]
    y = _ln_in(hs, s_ref[...], b_ref[...])
    qkv = _dot(y, wqkv_ref[...])
    col = lax.broadcasted_iota(jnp.int32, (LP, LP), 1)
    mask = jnp.where(col < L, 0.0, -1e9).astype(jnp.float32)
    pieces = []
    for h in range(H):
        q = qkv[:, h * DH:(h + 1) * DH]
        k = qkv[:, C + h * DH:C + (h + 1) * DH]
        v = qkv[:, 2 * C + h * DH:2 * C + (h + 1) * DH]
        sc = _dot_t(q, k) * (1.0 / np.sqrt(float(DH))) + mask
        m = jnp.max(sc, axis=-1, keepdims=True)
        e = jnp.exp(sc - m)
        p = e / jnp.sum(e, axis=-1, keepdims=True)
        pieces.append(_dot(p, v))
    o = jnp.concatenate(pieces, axis=1)
    out_ref[0] = hs + _dot(o, wo_ref[...])


def _router_kernel(h_ref, s_ref, b_ref, wr_ref, su_ref, y2_ref, cmb_ref):
    h = h_ref[...]
    y2 = _ln_in(h, s_ref[...], b_ref[...])
    y2_ref[...] = y2
    logits = _dot(y2, wr_ref[...])
    col = lax.broadcasted_iota(jnp.int32, (NTOK, 128), 1)
    logits = jnp.where(col < NEXP, logits, -1e30)
    mx = jnp.max(logits, axis=-1, keepdims=True)
    ex = jnp.exp(logits - mx)
    probs = ex / jnp.sum(ex, axis=-1, keepdims=True)
    su = su_ref[...]
    m1 = jnp.max(probs, axis=-1, keepdims=True)
    eq1 = (probs == m1).astype(jnp.float32)
    first = eq1 * (1.0 - jnp.minimum(_dot(eq1, su), 1.0))
    probs2 = probs - first * 2.0
    m2 = jnp.max(probs2, axis=-1, keepdims=True)
    eq2 = (probs2 == m2).astype(jnp.float32)
    second = eq2 * (1.0 - jnp.minimum(_dot(eq2, su), 1.0))
    denom = m1 + m2
    cmb = first * (m1 / denom) + second * (m2 / denom)
    row = lax.broadcasted_iota(jnp.int32, (NTOK, 128), 0)
    valid = (row % LP) < L
    cmb_ref[...] = jnp.where(valid, cmb, 0.0)


def _moe_dense_kernel(y2_ref, cmb_ref, h_ref, we1_ref, we2_ref, out_ref):
    e = pl.program_id(0)

    @pl.when(e == 0)
    def _():
        out_ref[...] = h_ref[...]

    a = jax.nn.gelu(_dot(y2_ref[...], we1_ref[0]))
    o = _dot(a, we2_ref[0])
    col = lax.broadcasted_iota(jnp.int32, (NTOK, 128), 1)
    w = jnp.sum(jnp.where(col == e, cmb_ref[...], 0.0), axis=-1, keepdims=True)
    out_ref[...] += o * w


def _head_kernel(clst_ref, wcls_ref, bcls_ref, cat_ref, m_ref, out_ref):
    proj = _dot(clst_ref[...], wcls_ref[...]) + bcls_ref[...]
    pn = proj / (jnp.sqrt(jnp.sum(proj * proj, axis=-1, keepdims=True)) + 1e-8)
    ct = cat_ref[...]
    cn = ct / (jnp.sqrt(jnp.sum(ct * ct, axis=-1, keepdims=True)) + 1e-8)
    sim = _dot_t(pn, cn)
    out_ref[...] = _dot(m_ref[...], sim)


@jax.jit
def _run(x_enc, W_patch, W_seq, pos_emb, ln1_s, ln1_b, Wqkv, Wo,
         ln2_s, ln2_b, Wr, We1, We2, Wcls, bcls, cat_tok):
    f32 = jnp.float32
    xt = jnp.transpose(x_enc, (0, 2, 1)).reshape(NSEQ, T)
    patches = xt.reshape(NSEQ * NPATCH, P)
    pos_p = jnp.zeros((LP, C), f32).at[:L].set(pos_emb)

    h = pl.pallas_call(
        _tokenize_kernel,
        out_shape=jax.ShapeDtypeStruct((NTOK, C), f32),
    )(patches, xt, jnp.asarray(_DPC), jnp.asarray(_DPS), W_patch,
      jnp.asarray(_DSC), jnp.asarray(_DSS), W_seq, pos_p)

    su = jnp.asarray(np.triu(np.ones((128, 128), np.float32), 1))
    wr_p = jnp.zeros((C, 128), f32)
    full = lambda shp: pl.BlockSpec(shp, lambda s: (0,) * len(shp))

    for l in range(LAYERS):
        h3 = h.reshape(NSEQ, LP, C)
        h3 = pl.pallas_call(
            _attn_kernel,
            grid=(NSEQ,),
            in_specs=[
                pl.BlockSpec((1, LP, C), lambda s: (s, 0, 0)),
                full((1, C)), full((1, C)),
                full((C, 3 * C)), full((C, C)),
            ],
            out_specs=pl.BlockSpec((1, LP, C), lambda s: (s, 0, 0)),
            out_shape=jax.ShapeDtypeStruct((NSEQ, LP, C), f32),
        )(h3, ln1_s[l][None], ln1_b[l][None], Wqkv[l], Wo[l])
        h = h3.reshape(NTOK, C)

        y2, cmb = pl.pallas_call(
            _router_kernel,
            out_shape=[jax.ShapeDtypeStruct((NTOK, C), f32),
                       jax.ShapeDtypeStruct((NTOK, 128), f32)],
        )(h, ln2_s[l][None], ln2_b[l][None], wr_p.at[:, :NEXP].set(Wr[l]), su)

        h = pl.pallas_call(
            _moe_dense_kernel,
            grid=(NEXP,),
            in_specs=[
                full((NTOK, C)), full((NTOK, 128)), full((NTOK, C)),
                pl.BlockSpec((1, C, DFF), lambda e: (e, 0, 0)),
                pl.BlockSpec((1, DFF, C), lambda e: (e, 0, 0)),
            ],
            out_specs=full((NTOK, C)),
            out_shape=jax.ShapeDtypeStruct((NTOK, C), f32),
        )(y2, cmb, h, We1[l], We2[l])

    clst = h.reshape(NSEQ, LP, C)[:, 0, :]
    cat_p = jnp.zeros((16, C), f32).at[:K].set(cat_tok)
    mmat = np.zeros((8, 16), np.float32)
    for b in range(B):
        mmat[b, b * V:(b + 1) * V] = 1.0 / V
    out = pl.pallas_call(
        _head_kernel,
        out_shape=jax.ShapeDtypeStruct((8, 16), f32),
    )(clst, Wcls, bcls[None], cat_p, jnp.asarray(mmat))
    return out[:B, :K]


def kernel(x_enc, x_mark_enc, W_patch, W_seq, pos_emb, ln1_s, ln1_b, Wqkv, Wo,
           ln2_s, ln2_b, Wr, We1, We2, Wcls, bcls, cat_tok):
    return _run(x_enc, W_patch, W_seq, pos_emb, ln1_s, ln1_b, Wqkv, Wo,
                ln2_s, ln2_b, Wr, We1, We2, Wcls, bcls, cat_tok)
